# baseline (speedup 1.0000x reference)
import dataclasses
import functools

import jax
import jax.numpy as jnp
from jax import lax
from jax.experimental import pallas as pl
from jax.experimental.pallas import tpu as pltpu
from jax.experimental.pallas import tpu_sc as plsc

_N = 20000
_K = 100
_PK = 128
_NBINS = _K * _PK
_DUMMY = _NBINS
_NCORES = 2
_NSUB = 16
_NTILES = _NCORES * _NSUB
_SLAB = 816
_SBINS = _NSUB * _SLAB
_NFLUSH = 10
_FLUSH = _NBINS // _NFLUSH
_PER_TILE = 640
_TAIL = _N - (_NTILES - 1) * _PER_TILE
_IOU_TH = 0.5
_LANES = 16
_CHUNK = 128
_NCHUNKS = _PER_TILE // _CHUNK

_vector_mesh = plsc.VectorSubcoreMesh(core_axis_name="c", subcore_axis_name="s")

_sc_params = pltpu.CompilerParams()
if "needs_layout_passes" in pltpu.CompilerParams.__dataclass_fields__:
    _sc_params = dataclasses.replace(_sc_params, needs_layout_passes=False)


@functools.partial(
    pl.kernel,
    out_type=jax.ShapeDtypeStruct((_NCORES * _NFLUSH, _FLUSH), jnp.int32),
    mesh=_vector_mesh,
    compiler_params=_sc_params,
    scratch_types=[
        pltpu.VMEM((_PER_TILE,), jnp.float32),
        pltpu.VMEM((_PER_TILE,), jnp.float32),
        [pltpu.VMEM((_CHUNK,), jnp.int32) for _ in range(_NCHUNKS)],
        pltpu.VMEM((_CHUNK,), jnp.int32),
        pltpu.VMEM((_SLAB,), jnp.int32),
        pltpu.VMEM_SHARED((_SBINS,), jnp.int32),
        pltpu.SemaphoreType.DMA,
        pltpu.SemaphoreType.DMA,
        pltpu.SemaphoreType.DMA,
    ],
)
def _sc_hist(pr_hbm, gt_hbm, out_hbm, pr_v, gt_v, idx_refs, ones_v, zslab_v,
             shared, add_sem, in_sem, z_sem):
    cid = lax.axis_index("c")
    sid = lax.axis_index("s")
    wid = sid * _NCORES + cid
    base = wid * _PER_TILE

    @pl.when(wid < _NTILES - 1)
    def _load_full():
        pltpu.async_copy(pr_hbm.at[pl.ds(base, _PER_TILE)], pr_v, in_sem)
        pltpu.async_copy(gt_hbm.at[pl.ds(base, _PER_TILE)], gt_v, in_sem)

    @pl.when(wid == _NTILES - 1)
    def _load_tail():
        pltpu.async_copy(pr_hbm.at[pl.ds(base, _TAIL)],
                         pr_v.at[pl.ds(0, _TAIL)], in_sem)
        pltpu.async_copy(gt_hbm.at[pl.ds(base, _TAIL)],
                         gt_v.at[pl.ds(0, _TAIL)], in_sem)

    zero16 = jnp.zeros((_LANES,), jnp.int32)
    one16 = jnp.ones((_LANES,), jnp.int32)
    for v in range(_SLAB // _LANES):
        zslab_v[pl.ds(v * _LANES, _LANES)] = zero16
    for v in range(_CHUNK // _LANES):
        ones_v[pl.ds(v * _LANES, _LANES)] = one16
    pltpu.async_copy(zslab_v, shared.at[pl.ds(sid * _SLAB, _SLAB)], z_sem)

    @pl.when(wid < _NTILES - 1)
    def _wait_full():
        pltpu.make_async_copy(
            pr_hbm.at[pl.ds(base, _PER_TILE)], pr_v, in_sem).wait()
        pltpu.make_async_copy(
            gt_hbm.at[pl.ds(base, _PER_TILE)], gt_v, in_sem).wait()

    @pl.when(wid == _NTILES - 1)
    def _wait_tail():
        pltpu.make_async_copy(
            pr_hbm.at[pl.ds(base, _TAIL)], pr_v.at[pl.ds(0, _TAIL)],
            in_sem).wait()
        pltpu.make_async_copy(
            gt_hbm.at[pl.ds(base, _TAIL)], gt_v.at[pl.ds(0, _TAIL)],
            in_sem).wait()

    for c in range(_NCHUNKS):
        for v in range(_CHUNK // _LANES):
            i = c * _CHUNK + v * _LANES
            pr = pr_v[pl.ds(i, _LANES)]
            gt = gt_v[pl.ds(i, _LANES)]
            pair = (pr * float(_PK) + gt).astype(jnp.int32)
            idx_refs[c][pl.ds(v * _LANES, _LANES)] = pair

    @pl.when(wid == _NTILES - 1)
    def _pad_tail():
        dummy16 = jnp.full((_LANES,), _DUMMY, jnp.int32)
        for c in range(_NCHUNKS):
            for v in range(_CHUNK // _LANES):
                i = c * _CHUNK + v * _LANES
                if i >= _TAIL:
                    idx_refs[c][pl.ds(v * _LANES, _LANES)] = dummy16

    pltpu.make_async_copy(
        zslab_v, shared.at[pl.ds(sid * _SLAB, _SLAB)], z_sem).wait()
    plsc.subcore_barrier()
    add_cps = [
        pltpu.async_copy(ones_v, shared.at[idx_refs[c]], add=True,
                         sem=add_sem)
        for c in range(_NCHUNKS)
    ]
    for cp in add_cps:
        cp.wait()
    plsc.subcore_barrier()

    @pl.when(sid < _NFLUSH)
    def _flush():
        pltpu.sync_copy(shared.at[pl.ds(sid * _FLUSH, _FLUSH)],
                        out_hbm.at[cid * _NFLUSH + sid])


def _ap_from_confusion(C):
    pr_counts = jnp.sum(C, axis=1, keepdims=True)
    gt_counts = jnp.sum(C, axis=0, keepdims=True)
    union = pr_counts + gt_counts - C
    iou = jnp.where(union > 0, C / jnp.maximum(union, 1.0), 0.0)
    pr_present = (pr_counts > 0).astype(jnp.float32)
    gt_present = (gt_counts > 0).astype(jnp.float32)
    tp = jnp.sum((iou >= _IOU_TH).astype(jnp.float32) * pr_present,
                 axis=0, keepdims=True)
    fp = jnp.sum(((iou > 0) & (iou < _IOU_TH)).astype(jnp.float32) * pr_present,
                 axis=0, keepdims=True)
    denom = tp + fp
    precision = jnp.where(denom > 0, tp / jnp.maximum(denom, 1.0), 0.0)
    num = jnp.sum(precision * gt_present, axis=(0, 1), keepdims=True)
    cnt = jnp.sum(gt_present, axis=(0, 1), keepdims=True)
    return num / jnp.maximum(cnt, 1.0)


def _tc_post(hist_ref, out_ref):
    h = hist_ref[...].reshape(_NCORES, _K, _PK)
    C = jnp.sum(h, axis=0).astype(jnp.float32)
    out_ref[...] = _ap_from_confusion(C)


def kernel(input, target):
    hist = _sc_hist(input, target)
    ap = pl.pallas_call(
        _tc_post,
        out_shape=jax.ShapeDtypeStruct((1, 1), jnp.float32),
    )(hist)
    return ap[0, 0]


# speedup vs baseline: 1.0102x; 1.0102x over previous
import dataclasses
import functools

import jax
import jax.numpy as jnp
from jax import lax
from jax.experimental import pallas as pl
from jax.experimental.pallas import tpu as pltpu
from jax.experimental.pallas import tpu_sc as plsc

_N = 20000
_K = 100
_PK = 128
_NBINS = _K * _PK
_DUMMY = _NBINS
_NCORES = 2
_NSUB = 16
_NTILES = _NCORES * _NSUB
_SLAB = 816
_SBINS = _NSUB * _SLAB
_NFLUSH = 10
_FLUSH = _NBINS // _NFLUSH
_PER_TILE = 640
_TAIL = _N - (_NTILES - 1) * _PER_TILE
_IOU_TH = 0.5
_LANES = 16
_CHUNK = 128
_NCHUNKS = _PER_TILE // _CHUNK

_vector_mesh = plsc.VectorSubcoreMesh(core_axis_name="c", subcore_axis_name="s")

_sc_params = pltpu.CompilerParams()
if "needs_layout_passes" in pltpu.CompilerParams.__dataclass_fields__:
    _sc_params = dataclasses.replace(_sc_params, needs_layout_passes=False)


@functools.partial(
    pl.kernel,
    out_type=jax.ShapeDtypeStruct((_NCORES, _NBINS), jnp.int32),
    mesh=_vector_mesh,
    compiler_params=_sc_params,
    scratch_types=[
        pltpu.VMEM((_PER_TILE,), jnp.float32),
        pltpu.VMEM((_PER_TILE,), jnp.float32),
        [pltpu.VMEM((_CHUNK,), jnp.int32) for _ in range(_NCHUNKS)],
        pltpu.VMEM((_CHUNK,), jnp.int32),
        pltpu.VMEM((_SLAB,), jnp.int32),
        pltpu.VMEM_SHARED((_SBINS,), jnp.int32),
        pltpu.SemaphoreType.DMA,
        pltpu.SemaphoreType.DMA,
        pltpu.SemaphoreType.DMA,
    ],
)
def _sc_hist(pr_hbm, gt_hbm, out_hbm, pr_v, gt_v, idx_refs, ones_v, zslab_v,
             shared, add_sem, in_sem, z_sem):
    cid = lax.axis_index("c")
    sid = lax.axis_index("s")
    wid = sid * _NCORES + cid
    base = wid * _PER_TILE

    @pl.when(wid < _NTILES - 1)
    def _load_full():
        pltpu.async_copy(pr_hbm.at[pl.ds(base, _PER_TILE)], pr_v, in_sem)
        pltpu.async_copy(gt_hbm.at[pl.ds(base, _PER_TILE)], gt_v, in_sem)

    @pl.when(wid == _NTILES - 1)
    def _load_tail():
        pltpu.async_copy(pr_hbm.at[pl.ds(base, _TAIL)],
                         pr_v.at[pl.ds(0, _TAIL)], in_sem)
        pltpu.async_copy(gt_hbm.at[pl.ds(base, _TAIL)],
                         gt_v.at[pl.ds(0, _TAIL)], in_sem)

    zero16 = jnp.zeros((_LANES,), jnp.int32)
    one16 = jnp.ones((_LANES,), jnp.int32)
    for v in range(_SLAB // _LANES):
        zslab_v[pl.ds(v * _LANES, _LANES)] = zero16
    for v in range(_CHUNK // _LANES):
        ones_v[pl.ds(v * _LANES, _LANES)] = one16
    pltpu.async_copy(zslab_v, shared.at[pl.ds(sid * _SLAB, _SLAB)], z_sem)

    @pl.when(wid < _NTILES - 1)
    def _wait_full():
        pltpu.make_async_copy(
            pr_hbm.at[pl.ds(base, _PER_TILE)], pr_v, in_sem).wait()
        pltpu.make_async_copy(
            gt_hbm.at[pl.ds(base, _PER_TILE)], gt_v, in_sem).wait()

    @pl.when(wid == _NTILES - 1)
    def _wait_tail():
        pltpu.make_async_copy(
            pr_hbm.at[pl.ds(base, _TAIL)], pr_v.at[pl.ds(0, _TAIL)],
            in_sem).wait()
        pltpu.make_async_copy(
            gt_hbm.at[pl.ds(base, _TAIL)], gt_v.at[pl.ds(0, _TAIL)],
            in_sem).wait()

    for c in range(_NCHUNKS):
        for v in range(_CHUNK // _LANES):
            i = c * _CHUNK + v * _LANES
            pr = pr_v[pl.ds(i, _LANES)]
            gt = gt_v[pl.ds(i, _LANES)]
            pair = (pr * float(_PK) + gt).astype(jnp.int32)
            idx_refs[c][pl.ds(v * _LANES, _LANES)] = pair

    @pl.when(wid == _NTILES - 1)
    def _pad_tail():
        dummy16 = jnp.full((_LANES,), _DUMMY, jnp.int32)
        for c in range(_NCHUNKS):
            for v in range(_CHUNK // _LANES):
                i = c * _CHUNK + v * _LANES
                if i >= _TAIL:
                    idx_refs[c][pl.ds(v * _LANES, _LANES)] = dummy16

    pltpu.make_async_copy(
        zslab_v, shared.at[pl.ds(sid * _SLAB, _SLAB)], z_sem).wait()
    plsc.subcore_barrier()
    add_cps = [
        pltpu.async_copy(ones_v, shared.at[idx_refs[c]], add=True,
                         sem=add_sem)
        for c in range(_NCHUNKS)
    ]
    for cp in add_cps:
        cp.wait()
    plsc.subcore_barrier()

    @pl.when(sid == 0)
    def _flush():
        pltpu.sync_copy(shared.at[pl.ds(0, _NBINS)], out_hbm.at[cid])


def _ap_from_confusion(C):
    pr_counts = jnp.sum(C, axis=1, keepdims=True)
    gt_counts = jnp.sum(C, axis=0, keepdims=True)
    union = pr_counts + gt_counts - C
    iou = jnp.where(union > 0, C / jnp.maximum(union, 1.0), 0.0)
    pr_present = (pr_counts > 0).astype(jnp.float32)
    gt_present = (gt_counts > 0).astype(jnp.float32)
    tp = jnp.sum((iou >= _IOU_TH).astype(jnp.float32) * pr_present,
                 axis=0, keepdims=True)
    fp = jnp.sum(((iou > 0) & (iou < _IOU_TH)).astype(jnp.float32) * pr_present,
                 axis=0, keepdims=True)
    denom = tp + fp
    precision = jnp.where(denom > 0, tp / jnp.maximum(denom, 1.0), 0.0)
    num = jnp.sum(precision * gt_present, axis=(0, 1), keepdims=True)
    cnt = jnp.sum(gt_present, axis=(0, 1), keepdims=True)
    return num / jnp.maximum(cnt, 1.0)


def _tc_post(hist_ref, out_ref):
    h = hist_ref[...].reshape(_NCORES, _K, _PK)
    C = jnp.sum(h, axis=0).astype(jnp.float32)
    out_ref[...] = _ap_from_confusion(C)


def kernel(input, target):
    hist = _sc_hist(input, target)
    ap = pl.pallas_call(
        _tc_post,
        out_shape=jax.ShapeDtypeStruct((1, 1), jnp.float32),
    )(hist)
    return ap[0, 0]


# speedup vs baseline: 1.0115x; 1.0013x over previous
import dataclasses
import functools

import jax
import jax.numpy as jnp
from jax import lax
from jax.experimental import pallas as pl
from jax.experimental.pallas import tpu as pltpu
from jax.experimental.pallas import tpu_sc as plsc

_N = 20000
_K = 100
_PK = 128
_NBINS = _K * _PK
_DUMMY = _NBINS
_NCORES = 2
_NSUB = 16
_NTILES = _NCORES * _NSUB
_SLAB = 816
_SBINS = _NSUB * _SLAB
_NFLUSH = 10
_FLUSH = _NBINS // _NFLUSH
_PER_TILE = 640
_TAIL = _N - (_NTILES - 1) * _PER_TILE
_IOU_TH = 0.5
_LANES = 16
_CHUNK = 128
_NCHUNKS = _PER_TILE // _CHUNK

_vector_mesh = plsc.VectorSubcoreMesh(core_axis_name="c", subcore_axis_name="s")

_sc_params = pltpu.CompilerParams()
if "needs_layout_passes" in pltpu.CompilerParams.__dataclass_fields__:
    _sc_params = dataclasses.replace(_sc_params, needs_layout_passes=False)


@functools.partial(
    pl.kernel,
    out_type=jax.ShapeDtypeStruct((_NCORES, _NBINS), jnp.float32),
    mesh=_vector_mesh,
    compiler_params=_sc_params,
    scratch_types=[
        pltpu.VMEM((_PER_TILE,), jnp.float32),
        pltpu.VMEM((_PER_TILE,), jnp.float32),
        [pltpu.VMEM((_CHUNK,), jnp.int32) for _ in range(_NCHUNKS)],
        pltpu.VMEM((_CHUNK,), jnp.float32),
        pltpu.VMEM((_SLAB,), jnp.float32),
        pltpu.VMEM_SHARED((_SBINS,), jnp.float32),
        pltpu.SemaphoreType.DMA,
        pltpu.SemaphoreType.DMA,
        pltpu.SemaphoreType.DMA,
    ],
)
def _sc_hist(pr_hbm, gt_hbm, out_hbm, pr_v, gt_v, idx_refs, ones_v, zslab_v,
             shared, add_sem, in_sem, z_sem):
    cid = lax.axis_index("c")
    sid = lax.axis_index("s")
    wid = sid * _NCORES + cid
    base = wid * _PER_TILE

    @pl.when(wid < _NTILES - 1)
    def _load_full():
        pltpu.async_copy(pr_hbm.at[pl.ds(base, _PER_TILE)], pr_v, in_sem)
        pltpu.async_copy(gt_hbm.at[pl.ds(base, _PER_TILE)], gt_v, in_sem)

    @pl.when(wid == _NTILES - 1)
    def _load_tail():
        pltpu.async_copy(pr_hbm.at[pl.ds(base, _TAIL)],
                         pr_v.at[pl.ds(0, _TAIL)], in_sem)
        pltpu.async_copy(gt_hbm.at[pl.ds(base, _TAIL)],
                         gt_v.at[pl.ds(0, _TAIL)], in_sem)

    zero16 = jnp.zeros((_LANES,), jnp.float32)
    one16 = jnp.ones((_LANES,), jnp.float32)
    for v in range(_SLAB // _LANES):
        zslab_v[pl.ds(v * _LANES, _LANES)] = zero16
    for v in range(_CHUNK // _LANES):
        ones_v[pl.ds(v * _LANES, _LANES)] = one16
    pltpu.async_copy(zslab_v, shared.at[pl.ds(sid * _SLAB, _SLAB)], z_sem)

    @pl.when(wid < _NTILES - 1)
    def _wait_full():
        pltpu.make_async_copy(
            pr_hbm.at[pl.ds(base, _PER_TILE)], pr_v, in_sem).wait()
        pltpu.make_async_copy(
            gt_hbm.at[pl.ds(base, _PER_TILE)], gt_v, in_sem).wait()

    @pl.when(wid == _NTILES - 1)
    def _wait_tail():
        pltpu.make_async_copy(
            pr_hbm.at[pl.ds(base, _TAIL)], pr_v.at[pl.ds(0, _TAIL)],
            in_sem).wait()
        pltpu.make_async_copy(
            gt_hbm.at[pl.ds(base, _TAIL)], gt_v.at[pl.ds(0, _TAIL)],
            in_sem).wait()

    for c in range(_NCHUNKS):
        for v in range(_CHUNK // _LANES):
            i = c * _CHUNK + v * _LANES
            pr = pr_v[pl.ds(i, _LANES)]
            gt = gt_v[pl.ds(i, _LANES)]
            pair = (pr * float(_PK) + gt).astype(jnp.int32)
            idx_refs[c][pl.ds(v * _LANES, _LANES)] = pair

    @pl.when(wid == _NTILES - 1)
    def _pad_tail():
        dummy16 = jnp.full((_LANES,), _DUMMY, jnp.int32)
        for c in range(_NCHUNKS):
            for v in range(_CHUNK // _LANES):
                i = c * _CHUNK + v * _LANES
                if i >= _TAIL:
                    idx_refs[c][pl.ds(v * _LANES, _LANES)] = dummy16

    pltpu.make_async_copy(
        zslab_v, shared.at[pl.ds(sid * _SLAB, _SLAB)], z_sem).wait()
    plsc.subcore_barrier()
    add_cps = [
        pltpu.async_copy(ones_v, shared.at[idx_refs[c]], add=True,
                         sem=add_sem)
        for c in range(_NCHUNKS)
    ]
    for cp in add_cps:
        cp.wait()
    plsc.subcore_barrier()

    @pl.when(sid == 0)
    def _flush():
        pltpu.sync_copy(shared.at[pl.ds(0, _NBINS)], out_hbm.at[cid])


def _ap_from_confusion(C):
    pr_counts = jnp.sum(C, axis=1, keepdims=True)
    gt_counts = jnp.sum(C, axis=0, keepdims=True)
    union = pr_counts + gt_counts - C
    iou = jnp.where(union > 0, C / jnp.maximum(union, 1.0), 0.0)
    pr_present = (pr_counts > 0).astype(jnp.float32)
    gt_present = (gt_counts > 0).astype(jnp.float32)
    tp = jnp.sum((iou >= _IOU_TH).astype(jnp.float32) * pr_present,
                 axis=0, keepdims=True)
    fp = jnp.sum(((iou > 0) & (iou < _IOU_TH)).astype(jnp.float32) * pr_present,
                 axis=0, keepdims=True)
    denom = tp + fp
    precision = jnp.where(denom > 0, tp / jnp.maximum(denom, 1.0), 0.0)
    num = jnp.sum(precision * gt_present, axis=(0, 1), keepdims=True)
    cnt = jnp.sum(gt_present, axis=(0, 1), keepdims=True)
    return num / jnp.maximum(cnt, 1.0)


def _tc_post(hist_ref, out_ref):
    h = hist_ref[...].reshape(_NCORES, _K, _PK)
    C = jnp.sum(h, axis=0)
    out_ref[...] = _ap_from_confusion(C)


def kernel(input, target):
    hist = _sc_hist(input, target)
    ap = pl.pallas_call(
        _tc_post,
        out_shape=jax.ShapeDtypeStruct((1, 1), jnp.float32),
    )(hist)
    return ap[0, 0]
